# speedup vs baseline: 12.0322x; 12.0322x over previous
import jax
import jax.numpy as jnp
from jax.experimental import pallas as pl
from jax.experimental.pallas import tpu as pltpu

_B = 64


def _gap_kernel(seg_ref, x_ref, gw_ref, gb_ref, fw_ref, fb_ref, out_ref,
                acc_ref, m_ref, d_ref):
    k = pl.program_id(0)
    nb = pl.num_programs(0)

    @pl.when(k == 0)
    def _init():
        acc_ref[...] = jnp.zeros_like(acc_ref)
        m_ref[...] = jnp.full_like(m_ref, -jnp.inf)
        d_ref[...] = jnp.zeros_like(d_ref)

    x = x_ref[...]
    seg = seg_ref[0]
    bn = x.shape[0]

    g_col = jnp.dot(x, gw_ref[...], preferred_element_type=jnp.float32)
    g_col = g_col + gb_ref[...]
    g_row = jnp.transpose(g_col)

    iota_b = jax.lax.broadcasted_iota(jnp.int32, (_B, bn), 0)
    onehot = iota_b == seg
    m_blk = jnp.where(onehot, g_row, -jnp.inf)

    bm = jnp.max(m_blk, axis=1, keepdims=True)
    m_old = m_ref[...]
    m_new = jnp.maximum(m_old, bm)
    m_safe = jnp.where(m_new == -jnp.inf, 0.0, m_new)
    scale = jnp.exp(m_old - m_safe)
    e_blk = jnp.exp(m_blk - m_safe)

    d_ref[...] = d_ref[...] * scale + jnp.sum(e_blk, axis=1, keepdims=True)
    acc_ref[...] = acc_ref[...] * scale + jnp.dot(
        e_blk, x, preferred_element_type=jnp.float32)
    m_ref[...] = m_new

    @pl.when(k == nb - 1)
    def _fin():
        d = d_ref[...]
        inv = jnp.where(d > 0, 1.0 / d, 0.0)
        pooled = acc_ref[...] * inv
        out = jnp.dot(pooled, fw_ref[...],
                      preferred_element_type=jnp.float32) + fb_ref[...]
        out_ref[...] = jnp.where(d > 0, out, 0.0)


def kernel(x, segment_ids, gate_W, gate_b, feat_W, feat_b):
    n, d = x.shape
    bn = 2000
    while n % bn != 0 or bn % 8 != 0:
        bn //= 2
    nb = n // bn

    seg = segment_ids.astype(jnp.int32).reshape(nb, 1, bn)
    gb = gate_b.astype(jnp.float32).reshape(1, 1)
    fb = feat_b.astype(jnp.float32).reshape(1, d)

    out = pl.pallas_call(
        _gap_kernel,
        grid=(nb,),
        in_specs=[
            pl.BlockSpec((1, 1, bn), lambda k: (k, 0, 0)),
            pl.BlockSpec((bn, d), lambda k: (k, 0)),
            pl.BlockSpec((d, 1), lambda k: (0, 0)),
            pl.BlockSpec((1, 1), lambda k: (0, 0)),
            pl.BlockSpec((d, d), lambda k: (0, 0)),
            pl.BlockSpec((1, d), lambda k: (0, 0)),
        ],
        out_specs=pl.BlockSpec((_B, d), lambda k: (0, 0)),
        out_shape=jax.ShapeDtypeStruct((_B, d), jnp.float32),
        scratch_shapes=[
            pltpu.VMEM((_B, d), jnp.float32),
            pltpu.VMEM((_B, 1), jnp.float32),
            pltpu.VMEM((_B, 1), jnp.float32),
        ],
    )(seg, x, gate_W, gb, feat_W, fb)
    return out


# speedup vs baseline: 13.7428x; 1.1422x over previous
import jax
import jax.numpy as jnp
from jax.experimental import pallas as pl
from jax.experimental.pallas import tpu as pltpu

_B = 64


def _gap_kernel(seg_ref, x_ref, gw_ref, gb_ref, fw_ref, fb_ref, out_ref,
                acc_ref, m_ref, d_ref):
    k = pl.program_id(0)
    nb = pl.num_programs(0)

    @pl.when(k == 0)
    def _init():
        acc_ref[...] = jnp.zeros_like(acc_ref)
        m_ref[...] = jnp.full_like(m_ref, -jnp.inf)
        d_ref[...] = jnp.zeros_like(d_ref)

    x = x_ref[...]
    seg = seg_ref[0]
    bn = x.shape[0]

    g_row = jax.lax.dot_general(
        gw_ref[...], x, (((0,), (1,)), ((), ())),
        preferred_element_type=jnp.float32) + gb_ref[...]

    iota_b = jax.lax.broadcasted_iota(jnp.int32, (_B, bn), 0)
    onehot = iota_b == seg
    m_blk = jnp.where(onehot, g_row, -jnp.inf)

    bm = jnp.max(m_blk, axis=1, keepdims=True)
    m_old = m_ref[...]
    m_new = jnp.maximum(m_old, bm)
    m_safe = jnp.where(m_new == -jnp.inf, 0.0, m_new)
    scale = jnp.exp(m_old - m_safe)
    e_blk = jnp.exp(m_blk - m_safe)

    d_ref[...] = d_ref[...] * scale + jnp.sum(e_blk, axis=1, keepdims=True)
    acc_ref[...] = acc_ref[...] * scale + jnp.dot(
        e_blk, x, preferred_element_type=jnp.float32)
    m_ref[...] = m_new

    @pl.when(k == nb - 1)
    def _fin():
        d = d_ref[...]
        inv = jnp.where(d > 0, 1.0 / d, 0.0)
        pooled = acc_ref[...] * inv
        out = jnp.dot(pooled, fw_ref[...],
                      preferred_element_type=jnp.float32) + fb_ref[...]
        out_ref[...] = jnp.where(d > 0, out, 0.0)


def kernel(x, segment_ids, gate_W, gate_b, feat_W, feat_b):
    n, d = x.shape
    bn = 2000
    while n % bn != 0 or bn % 8 != 0:
        bn //= 2
    nb = n // bn

    seg = segment_ids.astype(jnp.int32).reshape(nb, 1, bn)
    gb = gate_b.astype(jnp.float32).reshape(1, 1)
    fb = feat_b.astype(jnp.float32).reshape(1, d)

    out = pl.pallas_call(
        _gap_kernel,
        grid=(nb,),
        in_specs=[
            pl.BlockSpec((1, 1, bn), lambda k: (k, 0, 0)),
            pl.BlockSpec((bn, d), lambda k: (k, 0)),
            pl.BlockSpec((d, 1), lambda k: (0, 0)),
            pl.BlockSpec((1, 1), lambda k: (0, 0)),
            pl.BlockSpec((d, d), lambda k: (0, 0)),
            pl.BlockSpec((1, d), lambda k: (0, 0)),
        ],
        out_specs=pl.BlockSpec((_B, d), lambda k: (0, 0)),
        out_shape=jax.ShapeDtypeStruct((_B, d), jnp.float32),
        scratch_shapes=[
            pltpu.VMEM((_B, d), jnp.float32),
            pltpu.VMEM((_B, 1), jnp.float32),
            pltpu.VMEM((_B, 1), jnp.float32),
        ],
    )(seg, x, gate_W, gb, feat_W, fb)
    return out
